# baseline (speedup 1.0000x reference)
import jax
import jax.numpy as jnp
from jax import lax
from jax.experimental import pallas as pl
from jax.experimental.pallas import tpu as pltpu
from jax.experimental.pallas import tpu_sc as plsc

NC = 2
NS = 16
NW = NC * NS
LANES = 16
BATCH = 16384
RANK = 32
BPW = BATCH // NW
QTR = 128
NQ = BPW // QTR


def _mf_body(uidx_hbm, sidx_hbm, utab_hbm, itab_hbm, out_hbm,
             idx_u, idx_s, bufs_u0, bufs_u1, bufs_s0, bufs_s1, out_v,
             sem_u0, sem_u1, sem_s0, sem_s1):
  wid = lax.axis_index("s") * NC + lax.axis_index("c")
  base = wid * BPW

  pltpu.sync_copy(uidx_hbm.at[pl.ds(base, BPW)], idx_u)
  pltpu.sync_copy(sidx_hbm.at[pl.ds(base, BPW)], idx_s)

  bufs_u = [bufs_u0, bufs_u1]
  bufs_s = [bufs_s0, bufs_s1]
  sems_u = [sem_u0, sem_u1]
  sems_s = [sem_s0, sem_s1]

  def issue(q, slot):
    bu, bs = bufs_u[slot], bufs_s[slot]
    su, ss = sems_u[slot], sems_s[slot]

    def issue_group(g, _):
      vu = idx_u[pl.ds(q * QTR + g * LANES, LANES)]
      vs = idx_s[pl.ds(q * QTR + g * LANES, LANES)]
      for j in range(LANES):
        r = g * LANES + j
        pltpu.make_async_copy(
            utab_hbm.at[pl.ds(vu[j], 1)], bu.at[pl.ds(r, 1)], su).start()
        pltpu.make_async_copy(
            itab_hbm.at[pl.ds(vs[j], 1)], bs.at[pl.ds(r, 1)], ss).start()
      return 0

    lax.fori_loop(0, QTR // LANES, issue_group, 0)

  lane = lax.broadcasted_iota(jnp.int32, (LANES,), 0)

  def compute(q, slot):
    bu, bs = bufs_u[slot], bufs_s[slot]
    pltpu.make_async_copy(utab_hbm.at[pl.ds(0, QTR)], bu, sems_u[slot]).wait()
    pltpu.make_async_copy(itab_hbm.at[pl.ds(0, QTR)], bs, sems_s[slot]).wait()

    @plsc.parallel_loop(0, QTR // LANES, unroll=1)
    def _(g):
      acc = jnp.zeros((LANES,), jnp.float32)
      for j in range(LANES):
        r = g * LANES + j
        t = (bu[r, pl.ds(0, LANES)] * bs[r, pl.ds(0, LANES)]
             + bu[r, pl.ds(LANES, LANES)] * bs[r, pl.ds(LANES, LANES)])
        acc = jnp.where(lane == j, jnp.sum(t), acc)
      out_v[pl.ds(q * QTR + g * LANES, LANES)] = acc

  issue(0, 0)
  for q in range(NQ):
    if q + 1 < NQ:
      issue(q + 1, (q + 1) % 2)
    compute(q, q % 2)

  pltpu.sync_copy(out_v, out_hbm.at[pl.ds(base, BPW)])


@jax.jit
def kernel(userIdx, servIdx, user_table, item_table):
  uidx = userIdx.astype(jnp.int32)
  sidx = servIdx.astype(jnp.int32)
  scale = jnp.where(uidx[0] < jnp.int32(2_000_000_000),
                    jnp.float32(1.0), jnp.float32(2.0))
  user_table = user_table * scale
  item_table = item_table * scale
  mesh = plsc.VectorSubcoreMesh(core_axis_name="c", subcore_axis_name="s",
                                num_cores=NC, num_subcores=NS)
  f = pl.kernel(
      _mf_body,
      out_type=jax.ShapeDtypeStruct((BATCH,), jnp.float32),
      mesh=mesh,
      scratch_types=[
          pltpu.VMEM((BPW,), jnp.int32),
          pltpu.VMEM((BPW,), jnp.int32),
          pltpu.VMEM((QTR, RANK), jnp.float32),
          pltpu.VMEM((QTR, RANK), jnp.float32),
          pltpu.VMEM((QTR, RANK), jnp.float32),
          pltpu.VMEM((QTR, RANK), jnp.float32),
          pltpu.VMEM((BPW,), jnp.float32),
          pltpu.SemaphoreType.DMA,
          pltpu.SemaphoreType.DMA,
          pltpu.SemaphoreType.DMA,
          pltpu.SemaphoreType.DMA,
      ],
      compiler_params=pltpu.CompilerParams(
          needs_layout_passes=False, use_tc_tiling_on_sc=True),
  )
  return f(uidx, sidx, user_table, item_table)


# speedup vs baseline: 1.0496x; 1.0496x over previous
import jax
import jax.numpy as jnp
from jax import lax
from jax.experimental import pallas as pl
from jax.experimental.pallas import tpu as pltpu
from jax.experimental.pallas import tpu_sc as plsc

NC = 2
NS = 16
NW = NC * NS
LANES = 16
BATCH = 16384
RANK = 32
BPW = BATCH // NW
QTR = 128
NQ = BPW // QTR


def _mf_body(uidx_hbm, sidx_hbm, utab_hbm, itab_hbm, out_hbm,
             idx_u, idx_s, bufs_u0, bufs_u1, bufs_s0, bufs_s1, out_v,
             sem_u0, sem_u1, sem_s0, sem_s1):
  wid = lax.axis_index("s") * NC + lax.axis_index("c")
  base = wid * BPW

  pltpu.sync_copy(uidx_hbm.at[pl.ds(base, BPW)], idx_u)
  pltpu.sync_copy(sidx_hbm.at[pl.ds(base, BPW)], idx_s)

  bufs_u = [bufs_u0, bufs_u1]
  bufs_s = [bufs_s0, bufs_s1]
  sems_u = [sem_u0, sem_u1]
  sems_s = [sem_s0, sem_s1]

  def issue(q, slot):
    bu, bs = bufs_u[slot], bufs_s[slot]
    su, ss = sems_u[slot], sems_s[slot]

    def issue_group(g, _):
      vu = idx_u[pl.ds(q * QTR + g * LANES, LANES)]
      vs = idx_s[pl.ds(q * QTR + g * LANES, LANES)]
      for j in range(LANES):
        r = g * LANES + j
        pltpu.make_async_copy(
            utab_hbm.at[pl.ds(vu[j] * RANK, RANK)],
            bu.at[pl.ds(r * RANK, RANK)], su).start()
        pltpu.make_async_copy(
            itab_hbm.at[pl.ds(vs[j] * RANK, RANK)],
            bs.at[pl.ds(r * RANK, RANK)], ss).start()
      return 0

    lax.fori_loop(0, QTR // LANES, issue_group, 0)

  lane = lax.broadcasted_iota(jnp.int32, (LANES,), 0)

  def compute(q, slot):
    bu, bs = bufs_u[slot], bufs_s[slot]
    pltpu.make_async_copy(
        utab_hbm.at[pl.ds(0, QTR * RANK)], bu, sems_u[slot]).wait()
    pltpu.make_async_copy(
        itab_hbm.at[pl.ds(0, QTR * RANK)], bs, sems_s[slot]).wait()

    @plsc.parallel_loop(0, QTR // LANES, unroll=1)
    def _(g):
      acc = jnp.zeros((LANES,), jnp.float32)
      for j in range(LANES):
        r = (g * LANES + j) * RANK
        t = (bu[pl.ds(r, LANES)] * bs[pl.ds(r, LANES)]
             + bu[pl.ds(r + LANES, LANES)] * bs[pl.ds(r + LANES, LANES)])
        acc = jnp.where(lane == j, jnp.sum(t), acc)
      out_v[pl.ds(q * QTR + g * LANES, LANES)] = acc

  issue(0, 0)
  for q in range(NQ):
    if q + 1 < NQ:
      issue(q + 1, (q + 1) % 2)
    compute(q, q % 2)

  pltpu.sync_copy(out_v, out_hbm.at[pl.ds(base, BPW)])


@jax.jit
def kernel(userIdx, servIdx, user_table, item_table):
  uidx = userIdx.astype(jnp.int32)
  sidx = servIdx.astype(jnp.int32)
  utab = user_table.reshape(-1)
  itab = item_table.reshape(-1)
  mesh = plsc.VectorSubcoreMesh(core_axis_name="c", subcore_axis_name="s",
                                num_cores=NC, num_subcores=NS)
  f = pl.kernel(
      _mf_body,
      out_type=jax.ShapeDtypeStruct((BATCH,), jnp.float32),
      mesh=mesh,
      scratch_types=[
          pltpu.VMEM((BPW,), jnp.int32),
          pltpu.VMEM((BPW,), jnp.int32),
          pltpu.VMEM((QTR * RANK,), jnp.float32),
          pltpu.VMEM((QTR * RANK,), jnp.float32),
          pltpu.VMEM((QTR * RANK,), jnp.float32),
          pltpu.VMEM((QTR * RANK,), jnp.float32),
          pltpu.VMEM((BPW,), jnp.float32),
          pltpu.SemaphoreType.DMA,
          pltpu.SemaphoreType.DMA,
          pltpu.SemaphoreType.DMA,
          pltpu.SemaphoreType.DMA,
      ],
      compiler_params=pltpu.CompilerParams(needs_layout_passes=False),
  )
  return f(uidx, sidx, utab, itab)


# speedup vs baseline: 1.0853x; 1.0339x over previous
import jax
import jax.numpy as jnp
from jax import lax
from jax.experimental import pallas as pl
from jax.experimental.pallas import tpu as pltpu
from jax.experimental.pallas import tpu_sc as plsc

NC = 2
NS = 16
NW = NC * NS
LANES = 16
BATCH = 16384
RANK = 32
BPW = BATCH // NW
QTR = 128
NQ = BPW // QTR

FB = 2048
SUB = FB // 4


def _repack_block(tT_ref, out_ref):
  x = tT_ref[...]
  out_ref[...] = jnp.concatenate(
      [x[:, j * SUB:(j + 1) * SUB].T for j in range(4)], axis=1)


def _repack(tT, n_rows):
  grid = (n_rows + FB - 1) // FB
  return pl.pallas_call(
      _repack_block,
      grid=(grid,),
      in_specs=[pl.BlockSpec((RANK, FB), lambda c: (0, c))],
      out_specs=pl.BlockSpec((SUB, 4 * RANK), lambda c: (c, 0)),
      out_shape=jax.ShapeDtypeStruct((grid * SUB, 4 * RANK), jnp.float32),
  )(tT).reshape(-1)


def _mf_body(uidx_hbm, sidx_hbm, utab_hbm, itab_hbm, out_hbm,
             idx_u, idx_s, bufs_u0, bufs_u1, bufs_s0, bufs_s1, out_v,
             sem_u0, sem_u1, sem_s0, sem_s1):
  wid = lax.axis_index("s") * NC + lax.axis_index("c")
  base = wid * BPW

  pltpu.sync_copy(uidx_hbm.at[pl.ds(base, BPW)], idx_u)
  pltpu.sync_copy(sidx_hbm.at[pl.ds(base, BPW)], idx_s)

  bufs_u = [bufs_u0, bufs_u1]
  bufs_s = [bufs_s0, bufs_s1]
  sems_u = [sem_u0, sem_u1]
  sems_s = [sem_s0, sem_s1]

  def issue(q, slot):
    bu, bs = bufs_u[slot], bufs_s[slot]
    su, ss = sems_u[slot], sems_s[slot]

    def issue_group(g, _):
      vu = idx_u[pl.ds(q * QTR + g * LANES, LANES)]
      vs = idx_s[pl.ds(q * QTR + g * LANES, LANES)]
      vuo = ((vu >> 11) << 16) | ((vu & 511) << 7) | (((vu >> 9) & 3) << 5)
      vso = ((vs >> 11) << 16) | ((vs & 511) << 7) | (((vs >> 9) & 3) << 5)
      for j in range(LANES):
        r = g * LANES + j
        pltpu.make_async_copy(
            utab_hbm.at[pl.ds(pl.multiple_of(vuo[j], RANK), RANK)],
            bu.at[pl.ds(r * RANK, RANK)], su).start()
        pltpu.make_async_copy(
            itab_hbm.at[pl.ds(pl.multiple_of(vso[j], RANK), RANK)],
            bs.at[pl.ds(r * RANK, RANK)], ss).start()
      return 0

    lax.fori_loop(0, QTR // LANES, issue_group, 0)

  lane = lax.broadcasted_iota(jnp.int32, (LANES,), 0)

  def compute(q, slot):
    bu, bs = bufs_u[slot], bufs_s[slot]
    pltpu.make_async_copy(
        utab_hbm.at[pl.ds(0, QTR * RANK)], bu, sems_u[slot]).wait()
    pltpu.make_async_copy(
        itab_hbm.at[pl.ds(0, QTR * RANK)], bs, sems_s[slot]).wait()

    @plsc.parallel_loop(0, QTR // LANES, unroll=1)
    def _(g):
      acc = jnp.zeros((LANES,), jnp.float32)
      for j in range(LANES):
        r = g * LANES + j
        rr = r * RANK
        t = (bu[pl.ds(rr, LANES)] * bs[pl.ds(rr, LANES)]
             + bu[pl.ds(rr + LANES, LANES)] * bs[pl.ds(rr + LANES, LANES)])
        acc = jnp.where(lane == j, jnp.sum(t), acc)
      out_v[pl.ds(q * QTR + g * LANES, LANES)] = acc

  issue(0, 0)
  for q in range(NQ):
    if q + 1 < NQ:
      issue(q + 1, (q + 1) % 2)
    compute(q, q % 2)

  pltpu.sync_copy(out_v, out_hbm.at[pl.ds(base, BPW)])


@jax.jit
def kernel(userIdx, servIdx, user_table, item_table):
  uidx = userIdx.astype(jnp.int32)
  sidx = servIdx.astype(jnp.int32)
  utab = _repack(user_table.T, user_table.shape[0])
  itab = _repack(item_table.T, item_table.shape[0])
  mesh = plsc.VectorSubcoreMesh(core_axis_name="c", subcore_axis_name="s",
                                num_cores=NC, num_subcores=NS)
  f = pl.kernel(
      _mf_body,
      out_type=jax.ShapeDtypeStruct((BATCH,), jnp.float32),
      mesh=mesh,
      scratch_types=[
          pltpu.VMEM((BPW,), jnp.int32),
          pltpu.VMEM((BPW,), jnp.int32),
          pltpu.VMEM((QTR * RANK,), jnp.float32),
          pltpu.VMEM((QTR * RANK,), jnp.float32),
          pltpu.VMEM((QTR * RANK,), jnp.float32),
          pltpu.VMEM((QTR * RANK,), jnp.float32),
          pltpu.VMEM((BPW,), jnp.float32),
          pltpu.SemaphoreType.DMA,
          pltpu.SemaphoreType.DMA,
          pltpu.SemaphoreType.DMA,
          pltpu.SemaphoreType.DMA,
      ],
      compiler_params=pltpu.CompilerParams(needs_layout_passes=False),
  )
  return f(uidx, sidx, utab, itab)


# speedup vs baseline: 1.5152x; 1.3961x over previous
import jax
import jax.numpy as jnp
from jax import lax
from jax.experimental import pallas as pl
from jax.experimental.pallas import tpu as pltpu
from jax.experimental.pallas import tpu_sc as plsc

NC = 2
NS = 16
NW = NC * NS
LANES = 16
BATCH = 16384
RANK = 32
BPW = BATCH // NW
QTR = 128
NQ = BPW // QTR

FB = 8192
SUB = 512
NSL = FB // SUB


def _repack_block(tT_ref, out_ref):
  for j in range(NSL):
    out_ref[pl.ds((j // 4) * SUB, SUB), (j % 4) * RANK:(j % 4 + 1) * RANK] = (
        tT_ref[:, j * SUB:(j + 1) * SUB].T)


def _repack(tT, n_rows):
  grid = (n_rows + FB - 1) // FB
  return pl.pallas_call(
      _repack_block,
      grid=(grid,),
      in_specs=[pl.BlockSpec((RANK, FB), lambda c: (0, c))],
      out_specs=pl.BlockSpec((FB // 4, 4 * RANK), lambda c: (c, 0)),
      out_shape=jax.ShapeDtypeStruct((grid * (FB // 4), 4 * RANK), jnp.float32),
  )(tT).reshape(-1)


def _mf_body(uidx_hbm, sidx_hbm, utab_hbm, itab_hbm, out_hbm,
             idx_u, idx_s, bufs_u0, bufs_u1, bufs_s0, bufs_s1, out_v,
             sem_u0, sem_u1, sem_s0, sem_s1):
  wid = lax.axis_index("s") * NC + lax.axis_index("c")
  base = wid * BPW

  pltpu.sync_copy(uidx_hbm.at[pl.ds(base, BPW)], idx_u)
  pltpu.sync_copy(sidx_hbm.at[pl.ds(base, BPW)], idx_s)

  bufs_u = [bufs_u0, bufs_u1]
  bufs_s = [bufs_s0, bufs_s1]
  sems_u = [sem_u0, sem_u1]
  sems_s = [sem_s0, sem_s1]

  def issue(q, slot):
    bu, bs = bufs_u[slot], bufs_s[slot]
    su, ss = sems_u[slot], sems_s[slot]

    def issue_group(g, _):
      vu = idx_u[pl.ds(q * QTR + g * LANES, LANES)]
      vs = idx_s[pl.ds(q * QTR + g * LANES, LANES)]
      vuo = ((vu >> 11) << 16) | ((vu & 511) << 7) | (((vu >> 9) & 3) << 5)
      vso = ((vs >> 11) << 16) | ((vs & 511) << 7) | (((vs >> 9) & 3) << 5)
      for j in range(LANES):
        r = g * LANES + j
        pltpu.make_async_copy(
            utab_hbm.at[pl.ds(pl.multiple_of(vuo[j], RANK), RANK)],
            bu.at[pl.ds(r * RANK, RANK)], su).start()
        pltpu.make_async_copy(
            itab_hbm.at[pl.ds(pl.multiple_of(vso[j], RANK), RANK)],
            bs.at[pl.ds(r * RANK, RANK)], ss).start()
      return 0

    lax.fori_loop(0, QTR // LANES, issue_group, 0)

  lane = lax.broadcasted_iota(jnp.int32, (LANES,), 0)

  def compute(q, slot):
    bu, bs = bufs_u[slot], bufs_s[slot]
    pltpu.make_async_copy(
        utab_hbm.at[pl.ds(0, QTR * RANK)], bu, sems_u[slot]).wait()
    pltpu.make_async_copy(
        itab_hbm.at[pl.ds(0, QTR * RANK)], bs, sems_s[slot]).wait()

    @plsc.parallel_loop(0, QTR // LANES, unroll=1)
    def _(g):
      acc = jnp.zeros((LANES,), jnp.float32)
      for j in range(LANES):
        r = g * LANES + j
        rr = r * RANK
        t = (bu[pl.ds(rr, LANES)] * bs[pl.ds(rr, LANES)]
             + bu[pl.ds(rr + LANES, LANES)] * bs[pl.ds(rr + LANES, LANES)])
        acc = jnp.where(lane == j, jnp.sum(t), acc)
      out_v[pl.ds(q * QTR + g * LANES, LANES)] = acc

  issue(0, 0)
  for q in range(NQ):
    if q + 1 < NQ:
      issue(q + 1, (q + 1) % 2)
    compute(q, q % 2)

  pltpu.sync_copy(out_v, out_hbm.at[pl.ds(base, BPW)])


@jax.jit
def kernel(userIdx, servIdx, user_table, item_table):
  uidx = userIdx.astype(jnp.int32)
  sidx = servIdx.astype(jnp.int32)
  utab = _repack(user_table.T, user_table.shape[0])
  itab = _repack(item_table.T, item_table.shape[0])
  mesh = plsc.VectorSubcoreMesh(core_axis_name="c", subcore_axis_name="s",
                                num_cores=NC, num_subcores=NS)
  f = pl.kernel(
      _mf_body,
      out_type=jax.ShapeDtypeStruct((BATCH,), jnp.float32),
      mesh=mesh,
      scratch_types=[
          pltpu.VMEM((BPW,), jnp.int32),
          pltpu.VMEM((BPW,), jnp.int32),
          pltpu.VMEM((QTR * RANK,), jnp.float32),
          pltpu.VMEM((QTR * RANK,), jnp.float32),
          pltpu.VMEM((QTR * RANK,), jnp.float32),
          pltpu.VMEM((QTR * RANK,), jnp.float32),
          pltpu.VMEM((BPW,), jnp.float32),
          pltpu.SemaphoreType.DMA,
          pltpu.SemaphoreType.DMA,
          pltpu.SemaphoreType.DMA,
          pltpu.SemaphoreType.DMA,
      ],
      compiler_params=pltpu.CompilerParams(needs_layout_passes=False),
  )
  return f(uidx, sidx, utab, itab)


# speedup vs baseline: 1.5651x; 1.0329x over previous
import jax
import jax.numpy as jnp
from jax import lax
from jax.experimental import pallas as pl
from jax.experimental.pallas import tpu as pltpu
from jax.experimental.pallas import tpu_sc as plsc

NC = 2
NS = 16
NW = NC * NS
LANES = 16
BATCH = 16384
RANK = 32
BPW = BATCH // NW
QTR = 128
NQ = BPW // QTR

FB = 8192
SUB = 512
NSL = FB // SUB


def _repack_block(tT_ref, out_ref):
  for j in range(NSL):
    out_ref[pl.ds(j * SUB, SUB), 0:RANK] = tT_ref[:, j * SUB:(j + 1) * SUB].T


def _repack(tT, n_rows):
  grid = (n_rows + FB - 1) // FB
  return pl.pallas_call(
      _repack_block,
      grid=(grid,),
      in_specs=[pl.BlockSpec((RANK, FB), lambda c: (0, c))],
      out_specs=pl.BlockSpec((FB, 4 * RANK), lambda c: (c, 0)),
      out_shape=jax.ShapeDtypeStruct((grid * FB, 4 * RANK), jnp.float32),
  )(tT).reshape(-1)


def _mf_body(uidx_hbm, sidx_hbm, utab_hbm, itab_hbm, out_hbm,
             idx_u, idx_s, bufs_u0, bufs_u1, bufs_s0, bufs_s1, out_v,
             sem_u0, sem_u1, sem_s0, sem_s1):
  wid = lax.axis_index("s") * NC + lax.axis_index("c")
  base = wid * BPW

  pltpu.sync_copy(uidx_hbm.at[pl.ds(base, BPW)], idx_u)
  pltpu.sync_copy(sidx_hbm.at[pl.ds(base, BPW)], idx_s)

  bufs_u = [bufs_u0, bufs_u1]
  bufs_s = [bufs_s0, bufs_s1]
  sems_u = [sem_u0, sem_u1]
  sems_s = [sem_s0, sem_s1]

  def issue(q, slot):
    bu, bs = bufs_u[slot], bufs_s[slot]
    su, ss = sems_u[slot], sems_s[slot]

    def issue_group(g, _):
      vu = idx_u[pl.ds(q * QTR + g * LANES, LANES)]
      vs = idx_s[pl.ds(q * QTR + g * LANES, LANES)]
      vuo = vu << 7
      vso = vs << 7
      for j in range(LANES):
        r = g * LANES + j
        pltpu.make_async_copy(
            utab_hbm.at[pl.ds(pl.multiple_of(vuo[j], RANK), RANK)],
            bu.at[pl.ds(r * RANK, RANK)], su).start()
        pltpu.make_async_copy(
            itab_hbm.at[pl.ds(pl.multiple_of(vso[j], RANK), RANK)],
            bs.at[pl.ds(r * RANK, RANK)], ss).start()
      return 0

    lax.fori_loop(0, QTR // LANES, issue_group, 0)

  lane = lax.broadcasted_iota(jnp.int32, (LANES,), 0)

  def compute(q, slot):
    bu, bs = bufs_u[slot], bufs_s[slot]
    pltpu.make_async_copy(
        utab_hbm.at[pl.ds(0, QTR * RANK)], bu, sems_u[slot]).wait()
    pltpu.make_async_copy(
        itab_hbm.at[pl.ds(0, QTR * RANK)], bs, sems_s[slot]).wait()

    @plsc.parallel_loop(0, QTR // LANES, unroll=1)
    def _(g):
      acc = jnp.zeros((LANES,), jnp.float32)
      for j in range(LANES):
        r = g * LANES + j
        rr = r * RANK
        t = (bu[pl.ds(rr, LANES)] * bs[pl.ds(rr, LANES)]
             + bu[pl.ds(rr + LANES, LANES)] * bs[pl.ds(rr + LANES, LANES)])
        acc = jnp.where(lane == j, jnp.sum(t), acc)
      out_v[pl.ds(q * QTR + g * LANES, LANES)] = acc

  issue(0, 0)
  for q in range(NQ):
    if q + 1 < NQ:
      issue(q + 1, (q + 1) % 2)
    compute(q, q % 2)

  pltpu.sync_copy(out_v, out_hbm.at[pl.ds(base, BPW)])


@jax.jit
def kernel(userIdx, servIdx, user_table, item_table):
  uidx = userIdx.astype(jnp.int32)
  sidx = servIdx.astype(jnp.int32)
  utab = _repack(user_table.T, user_table.shape[0])
  itab = _repack(item_table.T, item_table.shape[0])
  mesh = plsc.VectorSubcoreMesh(core_axis_name="c", subcore_axis_name="s",
                                num_cores=NC, num_subcores=NS)
  f = pl.kernel(
      _mf_body,
      out_type=jax.ShapeDtypeStruct((BATCH,), jnp.float32),
      mesh=mesh,
      scratch_types=[
          pltpu.VMEM((BPW,), jnp.int32),
          pltpu.VMEM((BPW,), jnp.int32),
          pltpu.VMEM((QTR * RANK,), jnp.float32),
          pltpu.VMEM((QTR * RANK,), jnp.float32),
          pltpu.VMEM((QTR * RANK,), jnp.float32),
          pltpu.VMEM((QTR * RANK,), jnp.float32),
          pltpu.VMEM((BPW,), jnp.float32),
          pltpu.SemaphoreType.DMA,
          pltpu.SemaphoreType.DMA,
          pltpu.SemaphoreType.DMA,
          pltpu.SemaphoreType.DMA,
      ],
      compiler_params=pltpu.CompilerParams(needs_layout_passes=False),
  )
  return f(uidx, sidx, utab, itab)
